# initial kernel scaffold (unmeasured)
import jax
import jax.numpy as jnp
from jax import lax
from jax.experimental import pallas as pl
from jax.experimental.pallas import tpu as pltpu

N_DEV = 4


def kernel(x, w_mat, scale_x, scale_w):
    if x.dtype != jnp.float8_e4m3fn:
        x = x.astype(jnp.float8_e4m3fn)
    if w_mat.dtype != jnp.float8_e4m3fn:
        w_mat = w_mat.astype(jnp.float8_e4m3fn)
    scale = (scale_x.astype(jnp.float32) * scale_w.astype(jnp.float32)).reshape((1,))

    m, k = x.shape
    _, n = w_mat.shape

    def body(x_ref, w_ref, scale_ref, out_ref,
             xcom, wcom, xs_send, xs_recv, ws_send, ws_recv):
        my = lax.axis_index("i")
        left = lax.rem(my + N_DEV - 1, N_DEV)
        right = lax.rem(my + 1, N_DEV)

        barrier_sem = pltpu.get_barrier_semaphore()
        for nbr in (left, right):
            pl.semaphore_signal(
                barrier_sem, inc=1,
                device_id=(nbr,), device_id_type=pl.DeviceIdType.MESH,
            )
        pl.semaphore_wait(barrier_sem, 2)

        def dot32(a_ref, b_ref):
            return jnp.dot(a_ref[...], b_ref[...],
                           preferred_element_type=jnp.float32)

        xcom[0] = x_ref[...]
        wcom[0] = w_ref[...]

        for h in range(N_DEV - 1):
            s = h % 2
            r = (h + 1) % 2
            rx = pltpu.make_async_remote_copy(
                src_ref=xcom.at[s], dst_ref=xcom.at[r],
                send_sem=xs_send.at[s], recv_sem=xs_recv.at[r],
                device_id=(right,), device_id_type=pl.DeviceIdType.MESH,
            )
            rw = pltpu.make_async_remote_copy(
                src_ref=wcom.at[s], dst_ref=wcom.at[r],
                send_sem=ws_send.at[s], recv_sem=ws_recv.at[r],
                device_id=(right,), device_id_type=pl.DeviceIdType.MESH,
            )
            rx.start()
            rw.start()
            if h == 0:
                out_ref[...] = dot32(x_ref, w_ref)
            rx.wait()
            rw.wait()
            out_ref[...] += dot32(xcom.at[r], wcom.at[r])

        out_ref[...] *= scale_ref[0]

    return pl.pallas_call(
        body,
        out_shape=jax.ShapeDtypeStruct((m, n), jnp.float32),
        in_specs=[
            pl.BlockSpec(memory_space=pltpu.VMEM),
            pl.BlockSpec(memory_space=pltpu.VMEM),
            pl.BlockSpec(memory_space=pltpu.SMEM),
        ],
        out_specs=pl.BlockSpec(memory_space=pltpu.VMEM),
        scratch_shapes=[
            pltpu.VMEM((2, m, k), jnp.float8_e4m3fn),
            pltpu.VMEM((2, k, n), jnp.float8_e4m3fn),
            pltpu.SemaphoreType.DMA((2,)),
            pltpu.SemaphoreType.DMA((2,)),
            pltpu.SemaphoreType.DMA((2,)),
            pltpu.SemaphoreType.DMA((2,)),
        ],
        compiler_params=pltpu.CompilerParams(collective_id=0),
    )(x, w_mat, scale)


# baseline (device time: 294458 ns/iter reference)
import jax
import jax.numpy as jnp
from jax import lax
from jax.experimental import pallas as pl
from jax.experimental.pallas import tpu as pltpu

N_DEV = 4


def kernel(x, w_mat, scale_x, scale_w):
    if x.dtype != jnp.float8_e4m3fn:
        x = x.astype(jnp.float8_e4m3fn)
    if w_mat.dtype != jnp.float8_e4m3fn:
        w_mat = w_mat.astype(jnp.float8_e4m3fn)
    scale = (scale_x.astype(jnp.float32) * scale_w.astype(jnp.float32)).reshape((1,))

    m, k = x.shape
    _, n = w_mat.shape

    def body(x_ref, w_ref, scale_ref, out_ref,
             xcom, wcom, xs_send, xs_recv, ws_send, ws_recv):
        my = lax.axis_index("i")
        left = lax.rem(my + N_DEV - 1, N_DEV)
        right = lax.rem(my + 1, N_DEV)

        barrier_sem = pltpu.get_barrier_semaphore()
        for nbr in (left, right):
            pl.semaphore_signal(
                barrier_sem, inc=1,
                device_id=(nbr,), device_id_type=pl.DeviceIdType.MESH,
            )
        pl.semaphore_wait(barrier_sem, 2)

        def dot32(a_ref, b_ref):
            return jnp.dot(a_ref[...], b_ref[...],
                           preferred_element_type=jnp.float32)

        xcom[0] = x_ref[...]
        wcom[0] = w_ref[...]

        for h in range(N_DEV - 1):
            s = h % 2
            r = (h + 1) % 2
            rx = pltpu.make_async_remote_copy(
                src_ref=xcom.at[s], dst_ref=xcom.at[r],
                send_sem=xs_send.at[s], recv_sem=xs_recv.at[r],
                device_id=(right,), device_id_type=pl.DeviceIdType.MESH,
            )
            rw = pltpu.make_async_remote_copy(
                src_ref=wcom.at[s], dst_ref=wcom.at[r],
                send_sem=ws_send.at[s], recv_sem=ws_recv.at[r],
                device_id=(right,), device_id_type=pl.DeviceIdType.MESH,
            )
            rx.start()
            rw.start()
            if h == 0:
                out_ref[...] = dot32(x_ref, w_ref)
            rx.wait()
            rw.wait()
            out_ref[...] += dot32(xcom.at[r], wcom.at[r])

        out_ref[...] *= scale_ref[0]

    return pl.pallas_call(
        body,
        out_shape=jax.ShapeDtypeStruct((m, n), jnp.float32),
        in_specs=[
            pl.BlockSpec(memory_space=pltpu.VMEM),
            pl.BlockSpec(memory_space=pltpu.VMEM),
            pl.BlockSpec(memory_space=pltpu.SMEM),
        ],
        out_specs=pl.BlockSpec(memory_space=pltpu.VMEM),
        scratch_shapes=[
            pltpu.VMEM((2, m, k), jnp.float8_e4m3fn),
            pltpu.VMEM((2, k, n), jnp.float8_e4m3fn),
            pltpu.SemaphoreType.DMA((2,)),
            pltpu.SemaphoreType.DMA((2,)),
            pltpu.SemaphoreType.DMA((2,)),
            pltpu.SemaphoreType.DMA((2,)),
        ],
        compiler_params=pltpu.CompilerParams(
            collective_id=0,
            vmem_limit_bytes=128 * 1024 * 1024,
        ),
    )(x, w_mat, scale)


# device time: 176419 ns/iter; 1.6691x vs baseline; 1.6691x over previous
import jax
import jax.numpy as jnp
from jax import lax
from jax.experimental import pallas as pl
from jax.experimental.pallas import tpu as pltpu

N_DEV = 4


def kernel(x, w_mat, scale_x, scale_w):
    if x.dtype != jnp.float8_e4m3fn:
        x = x.astype(jnp.float8_e4m3fn)
    if w_mat.dtype != jnp.float8_e4m3fn:
        w_mat = w_mat.astype(jnp.float8_e4m3fn)
    scale = (scale_x.astype(jnp.float32) * scale_w.astype(jnp.float32)).reshape((1,))

    m, k = x.shape
    _, n = w_mat.shape
    mh = m // 2
    nh = n // 2

    def body(x_ref, w_ref, scale_ref, out_ref,
             xt_com, xb_com, wl_com, wr_com,
             xt_send, xt_recv, xb_send, xb_recv,
             wl_send, wl_recv, wr_send, wr_recv):
        my = lax.axis_index("i")
        left = lax.rem(my + N_DEV - 1, N_DEV)
        right = lax.rem(my + 1, N_DEV)

        barrier_sem = pltpu.get_barrier_semaphore()
        for nbr in (left, right):
            pl.semaphore_signal(
                barrier_sem, inc=1,
                device_id=(nbr,), device_id_type=pl.DeviceIdType.MESH,
            )
        pl.semaphore_wait(barrier_sem, 2)

        def rdma(src, dst, ssem, rsem, dev):
            return pltpu.make_async_remote_copy(
                src_ref=src, dst_ref=dst, send_sem=ssem, recv_sem=rsem,
                device_id=(dev,), device_id_type=pl.DeviceIdType.MESH,
            )

        def dot32(a, b):
            return jnp.dot(a, b, preferred_element_type=jnp.float32)

        TOP, BOT = pl.ds(0, mh), pl.ds(mh, mh)
        LFT, RGT = pl.ds(0, nh), pl.ds(nh, nh)

        for h in range(N_DEV - 1):
            if h == 0:
                xt_src = x_ref.at[TOP, :]
                xb_src = x_ref.at[BOT, :]
                wl_src = w_ref.at[:, LFT]
                wr_src = w_ref.at[:, RGT]
            else:
                xt_src = xt_com.at[h - 1]
                xb_src = xb_com.at[h - 1]
                wl_src = wl_com.at[h - 1]
                wr_src = wr_com.at[h - 1]
            hops = [
                rdma(xt_src, xt_com.at[h], xt_send.at[h], xt_recv.at[h], right),
                rdma(wl_src, wl_com.at[h], wl_send.at[h], wl_recv.at[h], right),
                rdma(xb_src, xb_com.at[h], xb_send.at[h], xb_recv.at[h], left),
                rdma(wr_src, wr_com.at[h], wr_send.at[h], wr_recv.at[h], left),
            ]
            for r in hops:
                r.start()
            if h == 0:
                out_ref[...] = dot32(x_ref[...], w_ref[...])
            elif h == 1:
                out_ref[TOP, LFT] += dot32(xt_com[0], wl_com[0])
                out_ref[BOT, RGT] += dot32(xb_com[0], wr_com[0])
            else:
                out_ref[TOP, LFT] += dot32(xt_com[1], wl_com[1])
                out_ref[BOT, RGT] += dot32(xb_com[1], wr_com[1])
                out_ref[TOP, RGT] += dot32(xt_com[1], wr_com[1])
                out_ref[BOT, LFT] += dot32(xb_com[1], wl_com[1])
            for r in hops:
                r.wait()

        out_ref[TOP, LFT] += dot32(xt_com[2], wl_com[2])
        out_ref[BOT, RGT] += dot32(xb_com[2], wr_com[2])
        out_ref[TOP, RGT] += dot32(xt_com[2], wr_com[0])
        out_ref[BOT, LFT] += dot32(xb_com[0], wl_com[2])
        out_ref[TOP, RGT] += dot32(xt_com[0], wr_com[2])
        out_ref[BOT, LFT] += dot32(xb_com[2], wl_com[0])

        out_ref[...] *= scale_ref[0]

    nhops = N_DEV - 1
    return pl.pallas_call(
        body,
        out_shape=jax.ShapeDtypeStruct((m, n), jnp.float32),
        in_specs=[
            pl.BlockSpec(memory_space=pltpu.VMEM),
            pl.BlockSpec(memory_space=pltpu.VMEM),
            pl.BlockSpec(memory_space=pltpu.SMEM),
        ],
        out_specs=pl.BlockSpec(memory_space=pltpu.VMEM),
        scratch_shapes=[
            pltpu.VMEM((nhops, mh, k), jnp.float8_e4m3fn),
            pltpu.VMEM((nhops, mh, k), jnp.float8_e4m3fn),
            pltpu.VMEM((nhops, k, nh), jnp.float8_e4m3fn),
            pltpu.VMEM((nhops, k, nh), jnp.float8_e4m3fn),
            pltpu.SemaphoreType.DMA((nhops,)),
            pltpu.SemaphoreType.DMA((nhops,)),
            pltpu.SemaphoreType.DMA((nhops,)),
            pltpu.SemaphoreType.DMA((nhops,)),
            pltpu.SemaphoreType.DMA((nhops,)),
            pltpu.SemaphoreType.DMA((nhops,)),
            pltpu.SemaphoreType.DMA((nhops,)),
            pltpu.SemaphoreType.DMA((nhops,)),
        ],
        compiler_params=pltpu.CompilerParams(
            collective_id=0,
            vmem_limit_bytes=128 * 1024 * 1024,
        ),
    )(x, w_mat, scale)
